# baseline (speedup 1.0000x reference)
import functools

import jax
import jax.numpy as jnp
from jax import lax
from jax.experimental import pallas as pl
from jax.experimental.pallas import tpu as pltpu
from jax.experimental.pallas import tpu_sc as plsc

B, C, Wf, H = 16, 4, 128, 512
D_C, D_S, K = 256, 64, 1024
D_IN = C * Wf
N_TOK = B * H
HB_SC = 2
HB_TC = B - HB_SC
N_SC = HB_SC * H

_SC_NC, _SC_NS = 2, 16
_NW = _SC_NC * _SC_NS
_TOK_PER_W = N_SC // _NW
_N_STREAMS = max(1, _TOK_PER_W // 8)
_ROWS_PER_STREAM = _TOK_PER_W // _N_STREAMS

_HBM = pl.BlockSpec(memory_space=pltpu.MemorySpace.HBM)


def _bdot(a, b):
    return jax.lax.dot_general(a, b, (((1,), (0,)), ((), ())),
                               preferred_element_type=jnp.float32)


def _tdot(a, b):
    return jax.lax.dot_general(a, b, (((0,), (1,)), ((), ())),
                               preferred_element_type=jnp.float32)


def _split(a):
    hi = a.astype(jnp.bfloat16)
    lo = (a - hi.astype(jnp.float32)).astype(jnp.bfloat16)
    return hi, lo


def _encode(xb, wc, ws, cbt):
    emb = jax.lax.dot_general(xb, wc, (((0,), (0,)), ((), ())),
                              preferred_element_type=jnp.float32)
    es = jnp.tanh(jax.lax.dot_general(xb, ws, (((0,), (0,)), ((), ())),
                                      preferred_element_type=jnp.float32))
    es_mean = jnp.mean(es, axis=0, keepdims=True)

    cross = jnp.dot(emb, cbt)
    tn = jnp.sum(emb * emb, axis=1, keepdims=True)
    cbn = jnp.sum(cbt * cbt, axis=0, keepdims=True)
    d2 = (tn - 2.0 * cross) + cbn

    m = jnp.min(d2, axis=1, keepdims=True)
    ii = jax.lax.broadcasted_iota(jnp.int32, (H, K), 1)
    idx = jnp.min(jnp.where(d2 == m, ii, jnp.int32(K)), axis=1,
                  keepdims=True)
    return emb, es_mean, m, ii, idx


def _decode(q, es_mean, wd):
    wd_hi, wd_lo = _split(wd)
    q_hi, q_lo = _split(q)
    es_h, es_l = _split(es_mean)
    pre = (_tdot(wd_hi[:D_C], q_hi) + _tdot(wd_lo[:D_C], q_hi)
           + _tdot(wd_hi[:D_C], q_lo))
    sv = (_tdot(wd_hi[D_C:], es_h) + _tdot(wd_lo[D_C:], es_h)
          + _tdot(wd_hi[D_C:], es_l))
    return jnp.tanh(pre + sv)


def _encode_kernel(xb_ref, wc_ref, ws_ref, cbt_ref,
                   emb_ref, idx_ref, part_ref, embs_ref):
    i = pl.program_id(0)
    emb, es_mean, m, _, idx = _encode(xb_ref[0], wc_ref[...], ws_ref[...],
                                      cbt_ref[...])
    emb_ref[0] = emb
    idx_ref[0] = idx
    embs_ref[pl.ds(i, 1), :] = es_mean
    part_ref[pl.ds(i, 1), :] = jnp.full((1, 128), jnp.sum(m), jnp.float32)


def _mono_kernel(xb_ref, wc_ref, ws_ref, cbt_ref, cb_ref, wd_ref, _emb_any,
                 emb_ref, idx_ref, part_ref, embs_ref, quant_ref, rec_ref):
    i = pl.program_id(0)
    emb, es_mean, m, ii, idx = _encode(xb_ref[0], wc_ref[...], ws_ref[...],
                                       cbt_ref[...])
    emb_ref[0] = emb
    idx_ref[0] = idx
    embs_ref[pl.ds(i, 1), :] = es_mean
    part_ref[pl.ds(i, 1), :] = jnp.full((1, 128), jnp.sum(m), jnp.float32)

    oh = (ii == idx).astype(jnp.bfloat16)
    cb_hi, cb_lo = _split(cb_ref[...])
    q = _bdot(oh, cb_hi) + _bdot(oh, cb_lo)
    quant_ref[0] = q
    rec_ref[0] = _decode(q, es_mean, wd_ref[...])


def _tail_kernel(q_ref, wd_ref, embs_ref, _q_any, _r_any, quant_ref, rec_ref):
    i = pl.program_id(0)
    q = q_ref[0]
    quant_ref[0] = q
    rec_ref[0] = _decode(q, embs_ref[pl.ds(i, 1), :], wd_ref[...])


_SC_MESH = plsc.VectorSubcoreMesh(core_axis_name="c", subcore_axis_name="s")


@functools.partial(
    pl.kernel, mesh=_SC_MESH,
    out_type=jax.ShapeDtypeStruct((N_SC, 2, 128), jnp.float32),
    scratch_types=[
        pltpu.VMEM((_TOK_PER_W,), jnp.int32),
        pltpu.VMEM((_TOK_PER_W, 2, 128), jnp.float32),
        pltpu.SemaphoreType.DMA,
    ],
)
def _sc_gather(table_hbm, idx_hbm, out_hbm, idx_v, rows_v, sem):
    wid = lax.axis_index("s") * _SC_NC + lax.axis_index("c")
    base = wid * _TOK_PER_W
    pltpu.sync_copy(idx_hbm.at[pl.ds(base, _TOK_PER_W)], idx_v)
    copies = []
    for s in range(_N_STREAMS):
        o = s * _ROWS_PER_STREAM
        copies.append(pltpu.async_copy(
            table_hbm.at[idx_v.at[pl.ds(o, _ROWS_PER_STREAM)]],
            rows_v.at[pl.ds(o, _ROWS_PER_STREAM)], sem))
    for c in copies:
        c.wait()
    pltpu.sync_copy(rows_v, out_hbm.at[pl.ds(base, _TOK_PER_W)])


def kernel(x, W_enc_c, W_enc_s, codebook, W_dec):
    xr = x.reshape(B, D_IN, H)
    cbT = codebook.T

    emb16, idx_a, part_a, embs_a = pl.pallas_call(
        _encode_kernel,
        grid=(HB_SC,),
        in_specs=[
            pl.BlockSpec((1, D_IN, H), lambda b: (b, 0, 0)),
            pl.BlockSpec((D_IN, D_C), lambda b: (0, 0)),
            pl.BlockSpec((D_IN, D_S), lambda b: (0, 0)),
            pl.BlockSpec((D_C, K), lambda b: (0, 0)),
        ],
        out_specs=[
            pl.BlockSpec((1, H, D_C), lambda b: (b, 0, 0)),
            pl.BlockSpec((1, H, 1), lambda b: (b, 0, 0)),
            pl.BlockSpec((HB_SC, 128), lambda b: (0, 0)),
            pl.BlockSpec((HB_SC, D_S), lambda b: (0, 0)),
        ],
        out_shape=[
            jax.ShapeDtypeStruct((B, H, D_C), jnp.float32),
            jax.ShapeDtypeStruct((HB_SC, H, 1), jnp.int32),
            jax.ShapeDtypeStruct((HB_SC, 128), jnp.float32),
            jax.ShapeDtypeStruct((HB_SC, D_S), jnp.float32),
        ],
    )(xr, W_enc_c, W_enc_s, cbT)

    qa = _sc_gather(codebook.reshape(K, 2, 128), idx_a.reshape(N_SC))

    emb16, idx_b, part_b, embs_b, quant16, rec16 = pl.pallas_call(
        _mono_kernel,
        grid=(HB_TC,),
        in_specs=[
            pl.BlockSpec((1, D_IN, H), lambda b: (b + HB_SC, 0, 0)),
            pl.BlockSpec((D_IN, D_C), lambda b: (0, 0)),
            pl.BlockSpec((D_IN, D_S), lambda b: (0, 0)),
            pl.BlockSpec((D_C, K), lambda b: (0, 0)),
            pl.BlockSpec((K, D_C), lambda b: (0, 0)),
            pl.BlockSpec((D_C + D_S, D_IN), lambda b: (0, 0)),
            _HBM,
        ],
        out_specs=[
            pl.BlockSpec((1, H, D_C), lambda b: (b + HB_SC, 0, 0)),
            pl.BlockSpec((1, H, 1), lambda b: (b, 0, 0)),
            pl.BlockSpec((HB_TC, 128), lambda b: (0, 0)),
            pl.BlockSpec((HB_TC, D_S), lambda b: (0, 0)),
            pl.BlockSpec((1, H, D_C), lambda b: (b + HB_SC, 0, 0)),
            pl.BlockSpec((1, D_IN, H), lambda b: (b + HB_SC, 0, 0)),
        ],
        out_shape=[
            jax.ShapeDtypeStruct((B, H, D_C), jnp.float32),
            jax.ShapeDtypeStruct((HB_TC, H, 1), jnp.int32),
            jax.ShapeDtypeStruct((HB_TC, 128), jnp.float32),
            jax.ShapeDtypeStruct((HB_TC, D_S), jnp.float32),
            jax.ShapeDtypeStruct((B, H, D_C), jnp.float32),
            jax.ShapeDtypeStruct((B, D_IN, H), jnp.float32),
        ],
        input_output_aliases={6: 0},
    )(xr, W_enc_c, W_enc_s, cbT, codebook, W_dec, emb16)

    quant16, rec16 = pl.pallas_call(
        _tail_kernel,
        grid=(HB_SC,),
        in_specs=[
            pl.BlockSpec((1, H, D_C), lambda b: (b, 0, 0)),
            pl.BlockSpec((D_C + D_S, D_IN), lambda b: (0, 0)),
            pl.BlockSpec((HB_SC, D_S), lambda b: (0, 0)),
            _HBM,
            _HBM,
        ],
        out_specs=[
            pl.BlockSpec((1, H, D_C), lambda b: (b, 0, 0)),
            pl.BlockSpec((1, D_IN, H), lambda b: (b, 0, 0)),
        ],
        out_shape=[
            jax.ShapeDtypeStruct((B, H, D_C), jnp.float32),
            jax.ShapeDtypeStruct((B, D_IN, H), jnp.float32),
        ],
        input_output_aliases={3: 0, 4: 1},
    )(qa.reshape(HB_SC, H, D_C), W_dec, embs_a, quant16, rec16)

    output = rec16.reshape(B, C, Wf, H)
    indices = jnp.concatenate([idx_a, idx_b], axis=0).reshape(B, H)
    commit_loss = ((jnp.sum(part_a[:, 0]) + jnp.sum(part_b[:, 0]))
                   / jnp.float32(N_TOK * D_C))
    emb_s = jnp.concatenate([embs_a, embs_b], axis=0)
    return (output, emb16, quant16, indices, commit_loss, emb_s)


# speedup vs baseline: 1.0525x; 1.0525x over previous
import functools

import jax
import jax.numpy as jnp
from jax import lax
from jax.experimental import pallas as pl
from jax.experimental.pallas import tpu as pltpu
from jax.experimental.pallas import tpu_sc as plsc

B, C, Wf, H = 16, 4, 128, 512
D_C, D_S, K = 256, 64, 1024
D_IN = C * Wf
N_TOK = B * H
HB_SC = 4
HB_TC = B - HB_SC
N_SC = HB_SC * H

_SC_NC, _SC_NS = 2, 16
_NW = _SC_NC * _SC_NS
_TOK_PER_W = N_SC // _NW
_N_STREAMS = max(1, _TOK_PER_W // 8)
_ROWS_PER_STREAM = _TOK_PER_W // _N_STREAMS

_HBM = pl.BlockSpec(memory_space=pltpu.MemorySpace.HBM)


def _bdot(a, b):
    return jax.lax.dot_general(a, b, (((1,), (0,)), ((), ())),
                               preferred_element_type=jnp.float32)


def _tdot(a, b):
    return jax.lax.dot_general(a, b, (((0,), (1,)), ((), ())),
                               preferred_element_type=jnp.float32)


def _split(a):
    hi = a.astype(jnp.bfloat16)
    lo = (a - hi.astype(jnp.float32)).astype(jnp.bfloat16)
    return hi, lo


def _encode(xb, wc, ws, cbt):
    emb = jax.lax.dot_general(xb, wc, (((0,), (0,)), ((), ())),
                              preferred_element_type=jnp.float32)
    es = jnp.tanh(jax.lax.dot_general(xb, ws, (((0,), (0,)), ((), ())),
                                      preferred_element_type=jnp.float32))
    es_mean = jnp.mean(es, axis=0, keepdims=True)

    cross = jnp.dot(emb, cbt)
    tn = jnp.sum(emb * emb, axis=1, keepdims=True)
    cbn = jnp.sum(cbt * cbt, axis=0, keepdims=True)
    d2 = (tn - 2.0 * cross) + cbn

    m = jnp.min(d2, axis=1, keepdims=True)
    ii = jax.lax.broadcasted_iota(jnp.int32, (H, K), 1)
    idx = jnp.min(jnp.where(d2 == m, ii, jnp.int32(K)), axis=1,
                  keepdims=True)
    return emb, es_mean, m, ii, idx


def _decode(q, es_mean, wd):
    wd_hi, wd_lo = _split(wd)
    q_hi, q_lo = _split(q)
    es_h, es_l = _split(es_mean)
    pre = (_tdot(wd_hi[:D_C], q_hi) + _tdot(wd_lo[:D_C], q_hi)
           + _tdot(wd_hi[:D_C], q_lo))
    sv = (_tdot(wd_hi[D_C:], es_h) + _tdot(wd_lo[D_C:], es_h)
          + _tdot(wd_hi[D_C:], es_l))
    return jnp.tanh(pre + sv)


def _encode_kernel(xb_ref, wc_ref, ws_ref, cbt_ref,
                   emb_ref, idx_ref, part_ref, embs_ref):
    i = pl.program_id(0)
    emb, es_mean, m, _, idx = _encode(xb_ref[0], wc_ref[...], ws_ref[...],
                                      cbt_ref[...])
    emb_ref[0] = emb
    idx_ref[0] = idx
    embs_ref[pl.ds(i, 1), :] = es_mean
    part_ref[pl.ds(i, 1), :] = jnp.full((1, 128), jnp.sum(m), jnp.float32)


def _mono_kernel(xb_ref, wc_ref, ws_ref, cbt_ref, cb_ref, wd_ref, _emb_any,
                 emb_ref, idx_ref, part_ref, embs_ref, quant_ref, rec_ref):
    i = pl.program_id(0)
    emb, es_mean, m, ii, idx = _encode(xb_ref[0], wc_ref[...], ws_ref[...],
                                       cbt_ref[...])
    emb_ref[0] = emb
    idx_ref[0] = idx
    embs_ref[pl.ds(i, 1), :] = es_mean
    part_ref[pl.ds(i, 1), :] = jnp.full((1, 128), jnp.sum(m), jnp.float32)

    oh = (ii == idx).astype(jnp.bfloat16)
    cb_hi, cb_lo = _split(cb_ref[...])
    q = _bdot(oh, cb_hi) + _bdot(oh, cb_lo)
    quant_ref[0] = q
    rec_ref[0] = _decode(q, es_mean, wd_ref[...])


def _tail_kernel(q_ref, wd_ref, embs_ref, _q_any, _r_any, quant_ref, rec_ref):
    i = pl.program_id(0)
    q = q_ref[0]
    quant_ref[0] = q
    rec_ref[0] = _decode(q, embs_ref[pl.ds(i, 1), :], wd_ref[...])


_SC_MESH = plsc.VectorSubcoreMesh(core_axis_name="c", subcore_axis_name="s")


@functools.partial(
    pl.kernel, mesh=_SC_MESH,
    out_type=jax.ShapeDtypeStruct((N_SC, 2, 128), jnp.float32),
    scratch_types=[
        pltpu.VMEM((_TOK_PER_W,), jnp.int32),
        pltpu.VMEM((_TOK_PER_W, 2, 128), jnp.float32),
        pltpu.SemaphoreType.DMA,
    ],
)
def _sc_gather(table_hbm, idx_hbm, out_hbm, idx_v, rows_v, sem):
    wid = lax.axis_index("s") * _SC_NC + lax.axis_index("c")
    base = wid * _TOK_PER_W
    pltpu.sync_copy(idx_hbm.at[pl.ds(base, _TOK_PER_W)], idx_v)
    copies = []
    for s in range(_N_STREAMS):
        o = s * _ROWS_PER_STREAM
        copies.append(pltpu.async_copy(
            table_hbm.at[idx_v.at[pl.ds(o, _ROWS_PER_STREAM)]],
            rows_v.at[pl.ds(o, _ROWS_PER_STREAM)], sem))
    for c in copies:
        c.wait()
    pltpu.sync_copy(rows_v, out_hbm.at[pl.ds(base, _TOK_PER_W)])


def kernel(x, W_enc_c, W_enc_s, codebook, W_dec):
    xr = x.reshape(B, D_IN, H)
    cbT = codebook.T

    emb16, idx_a, part_a, embs_a = pl.pallas_call(
        _encode_kernel,
        grid=(HB_SC,),
        in_specs=[
            pl.BlockSpec((1, D_IN, H), lambda b: (b, 0, 0)),
            pl.BlockSpec((D_IN, D_C), lambda b: (0, 0)),
            pl.BlockSpec((D_IN, D_S), lambda b: (0, 0)),
            pl.BlockSpec((D_C, K), lambda b: (0, 0)),
        ],
        out_specs=[
            pl.BlockSpec((1, H, D_C), lambda b: (b, 0, 0)),
            pl.BlockSpec((1, H, 1), lambda b: (b, 0, 0)),
            pl.BlockSpec((HB_SC, 128), lambda b: (0, 0)),
            pl.BlockSpec((HB_SC, D_S), lambda b: (0, 0)),
        ],
        out_shape=[
            jax.ShapeDtypeStruct((B, H, D_C), jnp.float32),
            jax.ShapeDtypeStruct((HB_SC, H, 1), jnp.int32),
            jax.ShapeDtypeStruct((HB_SC, 128), jnp.float32),
            jax.ShapeDtypeStruct((HB_SC, D_S), jnp.float32),
        ],
    )(xr, W_enc_c, W_enc_s, cbT)

    qa = _sc_gather(codebook.reshape(K, 2, 128), idx_a.reshape(N_SC))

    emb16, idx_b, part_b, embs_b, quant16, rec16 = pl.pallas_call(
        _mono_kernel,
        grid=(HB_TC,),
        in_specs=[
            pl.BlockSpec((1, D_IN, H), lambda b: (b + HB_SC, 0, 0)),
            pl.BlockSpec((D_IN, D_C), lambda b: (0, 0)),
            pl.BlockSpec((D_IN, D_S), lambda b: (0, 0)),
            pl.BlockSpec((D_C, K), lambda b: (0, 0)),
            pl.BlockSpec((K, D_C), lambda b: (0, 0)),
            pl.BlockSpec((D_C + D_S, D_IN), lambda b: (0, 0)),
            _HBM,
        ],
        out_specs=[
            pl.BlockSpec((1, H, D_C), lambda b: (b + HB_SC, 0, 0)),
            pl.BlockSpec((1, H, 1), lambda b: (b, 0, 0)),
            pl.BlockSpec((HB_TC, 128), lambda b: (0, 0)),
            pl.BlockSpec((HB_TC, D_S), lambda b: (0, 0)),
            pl.BlockSpec((1, H, D_C), lambda b: (b + HB_SC, 0, 0)),
            pl.BlockSpec((1, D_IN, H), lambda b: (b + HB_SC, 0, 0)),
        ],
        out_shape=[
            jax.ShapeDtypeStruct((B, H, D_C), jnp.float32),
            jax.ShapeDtypeStruct((HB_TC, H, 1), jnp.int32),
            jax.ShapeDtypeStruct((HB_TC, 128), jnp.float32),
            jax.ShapeDtypeStruct((HB_TC, D_S), jnp.float32),
            jax.ShapeDtypeStruct((B, H, D_C), jnp.float32),
            jax.ShapeDtypeStruct((B, D_IN, H), jnp.float32),
        ],
        input_output_aliases={6: 0},
    )(xr, W_enc_c, W_enc_s, cbT, codebook, W_dec, emb16)

    quant16, rec16 = pl.pallas_call(
        _tail_kernel,
        grid=(HB_SC,),
        in_specs=[
            pl.BlockSpec((1, H, D_C), lambda b: (b, 0, 0)),
            pl.BlockSpec((D_C + D_S, D_IN), lambda b: (0, 0)),
            pl.BlockSpec((HB_SC, D_S), lambda b: (0, 0)),
            _HBM,
            _HBM,
        ],
        out_specs=[
            pl.BlockSpec((1, H, D_C), lambda b: (b, 0, 0)),
            pl.BlockSpec((1, D_IN, H), lambda b: (b, 0, 0)),
        ],
        out_shape=[
            jax.ShapeDtypeStruct((B, H, D_C), jnp.float32),
            jax.ShapeDtypeStruct((B, D_IN, H), jnp.float32),
        ],
        input_output_aliases={3: 0, 4: 1},
    )(qa.reshape(HB_SC, H, D_C), W_dec, embs_a, quant16, rec16)

    output = rec16.reshape(B, C, Wf, H)
    indices = jnp.concatenate([idx_a, idx_b], axis=0).reshape(B, H)
    commit_loss = ((jnp.sum(part_a[:, 0]) + jnp.sum(part_b[:, 0]))
                   / jnp.float32(N_TOK * D_C))
    emb_s = jnp.concatenate([embs_a, embs_b], axis=0)
    return (output, emb16, quant16, indices, commit_loss, emb_s)
